# baseline (speedup 1.0000x reference)
import functools
import jax
import jax.numpy as jnp
from jax import lax
from jax.experimental import pallas as pl
from jax.experimental.pallas import tpu as pltpu
from jax.experimental.pallas import tpu_sc as plsc

B, N, D = 4, 8192, 768
NC, NS, L = 2, 16, 16
NW = NC * NS
PPW = N // NW
R = 16
NCH = PPW // R
NV = D // L
T = NCH * B
NB = 4


def _sc_add(inputs, pos_table):
    mesh = plsc.VectorSubcoreMesh(core_axis_name="c", subcore_axis_name="s")

    @functools.partial(
        pl.kernel,
        out_type=jax.ShapeDtypeStruct((B, N, D), jnp.float32),
        mesh=mesh,
        scratch_types=[
            pltpu.VMEM((2, R, D), jnp.float32),
            pltpu.VMEM((NB, R, D), jnp.float32),
            pltpu.SemaphoreType.DMA((2,)),
            pltpu.SemaphoreType.DMA((NB,)),
            pltpu.SemaphoreType.DMA((NB,)),
        ],
    )
    def k(inp_hbm, tab_hbm, out_hbm, tbuf, ibuf, tsem, lsem, ssem):
        wid = lax.axis_index("s") * NC + lax.axis_index("c")
        p_base = wid * PPW

        def start_load(c, b, s):
            pltpu.async_copy(
                inp_hbm.at[b, pl.ds(p_base + c * R, R)], ibuf.at[s], lsem.at[s])

        def wait_load(s):
            pltpu.make_async_copy(
                inp_hbm.at[0, pl.ds(0, R)], ibuf.at[s], lsem.at[s]).wait()

        def start_store(c, b, s):
            pltpu.async_copy(
                ibuf.at[s], out_hbm.at[b, pl.ds(p_base + c * R, R)], ssem.at[s])

        def wait_store(s):
            pltpu.make_async_copy(
                ibuf.at[s], out_hbm.at[0, pl.ds(0, R)], ssem.at[s]).wait()

        def start_tload(c, tk):
            pltpu.async_copy(
                tab_hbm.at[pl.ds(p_base + c * R, R)], tbuf.at[tk], tsem.at[tk])

        def wait_tload(tk):
            pltpu.make_async_copy(
                tab_hbm.at[pl.ds(0, R)], tbuf.at[0], tsem.at[tk]).wait()

        start_tload(0, 0)
        start_load(0, 0, 0)
        start_load(0, 1, 1)

        def round_(g, carry):
            for cc in range(2):
                c = 2 * g + cc
                for b in range(B):
                    it = (2 * g + cc) * B + b
                    s = b
                    s2 = (b + 2) % NB
                    c2 = c if b < 2 else c + 1
                    b2 = b + 2 if b < 2 else b - 2

                    @pl.when(it + 2 < T)
                    def _(c2=c2, b2=b2, s2=s2, it=it):
                        @pl.when(it + 2 >= NB)
                        def _():
                            wait_store(s2)
                        start_load(c2, b2, s2)

                    if b == 0:
                        wait_tload(cc)

                        @pl.when(c + 1 < NCH)
                        def _(c=c, cc=cc):
                            start_tload(c + 1, 1 - cc)

                    wait_load(s)
                    tb = tbuf.at[cc]
                    ib = ibuf.at[s]

                    def add_row(r2, carry3, tb=tb, ib=ib):
                        for rr in range(2):
                            r = 2 * r2 + rr
                            for j in range(NV):
                                plsc.addupdate(
                                    ib.at[r, pl.ds(j * L, L)], tb[r, pl.ds(j * L, L)])
                        return carry3

                    lax.fori_loop(0, R // 2, add_row, 0)
                    start_store(c, b, s)
            return carry

        lax.fori_loop(0, NCH // 2, round_, 0)
        for s in range(NB):
            wait_store(s)

    return k(inputs, pos_table)


def kernel(inputs, pos_table):
    return _sc_add(inputs, pos_table)
